# Initial kernel scaffold; baseline (speedup 1.0000x reference)
import jax
import jax.numpy as jnp
from jax.experimental import pallas as pl


def kernel(x, W1, W2, codebook):
    raise NotImplementedError("write your pallas kernel here")


# speedup vs baseline: 2.5695x; 2.5695x over previous
import jax
import jax.numpy as jnp
from jax.experimental import pallas as pl
from jax.experimental.pallas import tpu as pltpu

B = 32
F = 256
L = 160
CF = 2
K = 256
D = L * CF
T = 2048


def _vq_kernel(x_ref, w1_ref, w2_ref, cb_ref, out_ref,
               c0_ref, c1_ref, tab_ref, bias_ref):
    b = pl.program_id(0)

    @pl.when(b == 0)
    def _precompute():
        w2 = w2_ref[...].astype(jnp.bfloat16)
        cb = cb_ref[...]
        c0 = cb[:, :L].astype(jnp.bfloat16)
        c1 = cb[:, L:].astype(jnp.bfloat16)
        c0_ref[...] = c0
        c1_ref[...] = c1
        te = jax.lax.dot_general(w2, c0, (((1,), (1,)), ((), ())),
                                 preferred_element_type=jnp.float32)
        to = jax.lax.dot_general(w2, c1, (((1,), (1,)), ((), ())),
                                 preferred_element_type=jnp.float32)
        tab_ref[...] = jnp.concatenate([te, to], axis=1)
        bias_ref[...] = -0.5 * jnp.sum(cb * cb, axis=1, keepdims=True)

    xb = x_ref[0].astype(jnp.bfloat16)
    w1 = w1_ref[...].astype(jnp.bfloat16)
    z = jnp.dot(w1, xb, preferred_element_type=jnp.float32)
    zb = z.astype(jnp.bfloat16)
    u = jnp.dot(c0_ref[...], zb, preferred_element_type=jnp.float32)
    v = jnp.dot(c1_ref[...], zb, preferred_element_type=jnp.float32)
    s = u + jnp.roll(v, -1, axis=1) + bias_ref[...]
    smax = jnp.max(s, axis=0, keepdims=True)
    kio = jax.lax.broadcasted_iota(jnp.int32, (K, T), 0)
    idx = jnp.min(jnp.where(s == smax, kio, K), axis=0, keepdims=True)
    lane = jax.lax.broadcasted_iota(jnp.int32, (1, T), 1)
    parity = lane % 2
    idx_at = jnp.where(parity == 0, idx, jnp.roll(idx, 1, axis=1))
    target = idx_at + K * parity
    k2 = jax.lax.broadcasted_iota(jnp.int32, (2 * K, T), 0)
    onehot = (k2 == target).astype(jnp.bfloat16)
    tab = tab_ref[...]
    tab_hi = tab.astype(jnp.bfloat16)
    tab_lo = (tab - tab_hi.astype(jnp.float32)).astype(jnp.bfloat16)
    out_ref[0] = (jnp.dot(tab_hi, onehot, preferred_element_type=jnp.float32)
                  + jnp.dot(tab_lo, onehot, preferred_element_type=jnp.float32))


@jax.jit
def kernel(x, W1, W2, codebook):
    xs = x[..., 0]
    out = pl.pallas_call(
        _vq_kernel,
        grid=(B,),
        in_specs=[
            pl.BlockSpec((1, F, T), lambda b: (b, 0, 0)),
            pl.BlockSpec((L, F), lambda b: (0, 0)),
            pl.BlockSpec((F, L), lambda b: (0, 0)),
            pl.BlockSpec((K, D), lambda b: (0, 0)),
        ],
        out_specs=pl.BlockSpec((1, F, T), lambda b: (b, 0, 0)),
        out_shape=jax.ShapeDtypeStruct((B, F, T), jnp.float32),
        scratch_shapes=[
            pltpu.VMEM((K, L), jnp.bfloat16),
            pltpu.VMEM((K, L), jnp.bfloat16),
            pltpu.VMEM((F, 2 * K), jnp.float32),
            pltpu.VMEM((K, 1), jnp.float32),
        ],
    )(xs, W1, W2, codebook)
    return out[..., None]


# speedup vs baseline: 2.8070x; 1.0924x over previous
import jax
import jax.numpy as jnp
from jax.experimental import pallas as pl
from jax.experimental.pallas import tpu as pltpu

B = 32
F = 256
L = 160
CF = 2
K = 256
D = L * CF
T = 2048


def _vq_kernel(x_ref, w1_ref, w2_ref, cb_ref, out_ref,
               c0_ref, c1_ref, tab_ref, bias_ref):
    b = pl.program_id(0)

    @pl.when(b == 0)
    def _precompute():
        w2 = w2_ref[...].astype(jnp.bfloat16)
        cb = cb_ref[...]
        c0 = cb[:, :L].astype(jnp.bfloat16)
        c1 = cb[:, L:].astype(jnp.bfloat16)
        c0_ref[...] = c0
        c1_ref[...] = c1
        te = jax.lax.dot_general(w2, c0, (((1,), (1,)), ((), ())),
                                 preferred_element_type=jnp.float32)
        to = jax.lax.dot_general(w2, c1, (((1,), (1,)), ((), ())),
                                 preferred_element_type=jnp.float32)
        tab_ref[...] = jnp.concatenate([te, to], axis=1).astype(jnp.bfloat16)
        bias_ref[...] = -0.5 * jnp.sum(cb * cb, axis=1, keepdims=True)

    z = jnp.dot(w1_ref[...], x_ref[0],
                preferred_element_type=jnp.float32)
    zb = z.astype(jnp.bfloat16)
    u = jnp.dot(c0_ref[...], zb, preferred_element_type=jnp.float32)
    v = jnp.dot(c1_ref[...], zb, preferred_element_type=jnp.float32)
    s = u + jnp.roll(v, -1, axis=1) + bias_ref[...]
    smax = jnp.max(s, axis=0, keepdims=True)
    kio = jax.lax.broadcasted_iota(jnp.int32, (K, T), 0)
    idx = jnp.min(jnp.where(s == smax, kio, K), axis=0, keepdims=True)
    lane = jax.lax.broadcasted_iota(jnp.int32, (1, T), 1)
    parity = lane % 2
    idx_at = jnp.where(parity == 0, idx, jnp.roll(idx, 1, axis=1))
    target = idx_at + K * parity
    k2 = jax.lax.broadcasted_iota(jnp.int32, (2 * K, T), 0)
    onehot = (k2 == target).astype(jnp.bfloat16)
    out_ref[0] = jnp.dot(tab_ref[...], onehot,
                         preferred_element_type=jnp.float32)


@jax.jit
def kernel(x, W1, W2, codebook):
    xs = x[..., 0]
    out = pl.pallas_call(
        _vq_kernel,
        grid=(B,),
        in_specs=[
            pl.BlockSpec((1, F, T), lambda b: (b, 0, 0)),
            pl.BlockSpec((L, F), lambda b: (0, 0)),
            pl.BlockSpec((F, L), lambda b: (0, 0)),
            pl.BlockSpec((K, D), lambda b: (0, 0)),
        ],
        out_specs=pl.BlockSpec((1, F, T), lambda b: (b, 0, 0)),
        out_shape=jax.ShapeDtypeStruct((B, F, T), jnp.float32),
        scratch_shapes=[
            pltpu.VMEM((K, L), jnp.bfloat16),
            pltpu.VMEM((K, L), jnp.bfloat16),
            pltpu.VMEM((F, 2 * K), jnp.bfloat16),
            pltpu.VMEM((K, 1), jnp.float32),
        ],
    )(xs, W1, W2, codebook)
    return out[..., None]


# speedup vs baseline: 2.9293x; 1.0436x over previous
import jax
import jax.numpy as jnp
from jax.experimental import pallas as pl
from jax.experimental.pallas import tpu as pltpu

B = 32
F = 256
L = 160
CF = 2
K = 256
D = L * CF
T = 2048


def _vq_kernel(x_ref, w1_ref, w2_ref, cb_ref, out_ref,
               w1b_ref, c0_ref, c1_ref, te_ref, to_ref, bias_ref):
    b = pl.program_id(0)

    @pl.when(b == 0)
    def _precompute():
        w1b_ref[...] = w1_ref[...].astype(jnp.bfloat16)
        w2 = w2_ref[...].astype(jnp.bfloat16)
        cb = cb_ref[...]
        c0 = cb[:, :L].astype(jnp.bfloat16)
        c1 = cb[:, L:].astype(jnp.bfloat16)
        c0_ref[...] = c0
        c1_ref[...] = c1
        te = jax.lax.dot_general(w2, c0, (((1,), (1,)), ((), ())),
                                 preferred_element_type=jnp.float32)
        to = jax.lax.dot_general(w2, c1, (((1,), (1,)), ((), ())),
                                 preferred_element_type=jnp.float32)
        te_ref[...] = te.astype(jnp.bfloat16)
        to_ref[...] = to.astype(jnp.bfloat16)
        bias_ref[...] = -0.5 * jnp.sum(cb * cb, axis=1, keepdims=True)

    kio = jax.lax.broadcasted_iota(jnp.int32, (K, T), 0)
    lane = jax.lax.broadcasted_iota(jnp.int32, (1, T), 1)
    parity = lane % 2
    for i in range(4):
        xb = x_ref[i].astype(jnp.bfloat16)
        z = jnp.dot(w1b_ref[...], xb,
                    preferred_element_type=jnp.float32)
        zb = z.astype(jnp.bfloat16)
        u = jnp.dot(c0_ref[...], zb, preferred_element_type=jnp.float32)
        v = jnp.dot(c1_ref[...], zb, preferred_element_type=jnp.float32)
        s = u + jnp.roll(v, -1, axis=1) + bias_ref[...]
        smax = jnp.max(s, axis=0, keepdims=True)
        idx = jnp.min(jnp.where(s == smax, kio, K), axis=0, keepdims=True)
        idx_at = jnp.where(parity == 0, idx, jnp.roll(idx, 1, axis=1))
        onehot = (kio == idx_at).astype(jnp.bfloat16)
        oute = jnp.dot(te_ref[...], onehot, preferred_element_type=jnp.float32)
        outo = jnp.dot(to_ref[...], onehot, preferred_element_type=jnp.float32)
        out_ref[i] = jnp.where(parity == 0, oute, outo)


@jax.jit
def kernel(x, W1, W2, codebook):
    xs = x.reshape(B, F, T)
    out = pl.pallas_call(
        _vq_kernel,
        grid=(B // 4,),
        in_specs=[
            pl.BlockSpec((4, F, T), lambda b: (b, 0, 0)),
            pl.BlockSpec((L, F), lambda b: (0, 0)),
            pl.BlockSpec((F, L), lambda b: (0, 0)),
            pl.BlockSpec((K, D), lambda b: (0, 0)),
        ],
        out_specs=pl.BlockSpec((4, F, T), lambda b: (b, 0, 0)),
        out_shape=jax.ShapeDtypeStruct((B, F, T), jnp.float32),
        scratch_shapes=[
            pltpu.VMEM((L, F), jnp.bfloat16),
            pltpu.VMEM((K, L), jnp.bfloat16),
            pltpu.VMEM((K, L), jnp.bfloat16),
            pltpu.VMEM((F, K), jnp.bfloat16),
            pltpu.VMEM((F, K), jnp.bfloat16),
            pltpu.VMEM((K, 1), jnp.float32),
        ],
    )(xs, W1, W2, codebook)
    return out.reshape(B, F, T, 1)
